# baseline (device time: 17680 ns/iter reference)
import jax
import jax.numpy as jnp
from jax import lax
from jax.experimental import pallas as pl
from jax.experimental.pallas import tpu as pltpu

N_DEV = 8
B, SQ, H, D = 2, 128, 4, 64
HD = H * D
ROWS = B * SQ
PC = HD + H
NEG = -1e9
BF = jnp.bfloat16


def kernel(x, Wq, K_ext, V_ext, Wo):
    def body(x_ref, wq_ref, k_ref, v_ref, wo_ref, out_ref,
             acc_a, acc_b, recv_a, recv_b,
             send_a, recv_sem_a, send_b, recv_sem_b):
        my = lax.axis_index("i")
        p_x = jnp.bitwise_xor(my, 1)
        loc = lax.rem(my, 4)
        p_y = my - loc + (3 - loc)
        p_z = jnp.bitwise_xor(my, 4)
        order_a = [p_x, p_y, p_z]
        order_b = [p_y, p_z, p_x]

        barrier_sem = pltpu.get_barrier_semaphore()
        for nbr in (p_x, p_y, p_z):
            pl.semaphore_signal(
                barrier_sem, inc=1,
                device_id=(nbr,), device_id_type=pl.DeviceIdType.MESH,
            )
        pl.semaphore_wait(barrier_sem, 3)

        x2 = x_ref[...].reshape(ROWS, 512).astype(BF)
        q2 = (jnp.dot(x2, wq_ref[...].astype(BF),
                      preferred_element_type=jnp.float32)
              * 0.125).astype(BF)
        k2 = k_ref[...].reshape(ROWS, HD).astype(BF)
        v2 = v_ref[...].reshape(ROWS, HD).astype(BF)
        ones_col = jnp.ones((SQ, 1), BF)

        ri = lax.broadcasted_iota(jnp.int32, (SQ, SQ), 0) // 64
        ci = lax.broadcasted_iota(jnp.int32, (SQ, SQ), 1) // 64
        blockdiag = ri == ci
        is_even = lax.rem(my, 2) == 0

        def partial(b, acc):
            for hh in range(H):
                q = q2[b * SQ:(b + 1) * SQ, hh * D:(hh + 1) * D]
                kc = k2[b * SQ:(b + 1) * SQ, hh * D:(hh + 1) * D]
                vc = v2[b * SQ:(b + 1) * SQ, hh * D:(hh + 1) * D]
                s_t = lax.dot_general(
                    kc, q, (((1,), (1,)), ((), ())),
                    preferred_element_type=jnp.float32,
                )
                p_t = jnp.exp(jnp.where(blockdiag, s_t, NEG)).astype(BF)
                v_aug = jnp.concatenate([vc, ones_col], axis=1)
                o_aug = lax.dot_general(
                    p_t, v_aug, (((0,), (0,)), ((), ())),
                    preferred_element_type=jnp.float32,
                )
                o_aug = jnp.where(is_even, o_aug, 0.0).astype(BF)
                acc[0:SQ, hh * D:(hh + 1) * D] = o_aug[:, 0:D]
                acc[0:SQ, HD + hh:HD + hh + 1] = o_aug[:, D:D + 1]

        def stage(s, acc, recv, ssem, rsem, partner):
            return pltpu.make_async_remote_copy(
                src_ref=acc,
                dst_ref=recv.at[s],
                send_sem=ssem.at[s],
                recv_sem=rsem.at[s],
                device_id=(partner,),
                device_id_type=pl.DeviceIdType.MESH,
            )

        def finish(b, acc):
            o2 = acc[0:SQ, 0:HD].astype(jnp.float32)
            heads = []
            for hh in range(H):
                l_col = acc[0:SQ, HD + hh:HD + hh + 1].astype(jnp.float32)
                heads.append(o2[:, hh * D:(hh + 1) * D] / l_col)
            ctx = jnp.concatenate(heads, axis=1)
            out_b = jnp.dot(ctx.astype(BF), wo_ref[...].astype(BF),
                            preferred_element_type=jnp.float32)
            out_ref[b, :, :] = out_b

        partial(0, acc_a)
        a0 = stage(0, acc_a, recv_a, send_a, recv_sem_a, order_a[0])
        a0.start()
        partial(1, acc_b)
        b0 = stage(0, acc_b, recv_b, send_b, recv_sem_b, order_b[0])
        b0.start()

        a0.wait()
        acc_a[...] = acc_a[...] + recv_a[0, :, :]
        a1 = stage(1, acc_a, recv_a, send_a, recv_sem_a, order_a[1])
        a1.start()
        b0.wait()
        acc_b[...] = acc_b[...] + recv_b[0, :, :]
        b1 = stage(1, acc_b, recv_b, send_b, recv_sem_b, order_b[1])
        b1.start()

        a1.wait()
        acc_a[...] = acc_a[...] + recv_a[1, :, :]
        a2 = stage(2, acc_a, recv_a, send_a, recv_sem_a, order_a[2])
        a2.start()
        b1.wait()
        acc_b[...] = acc_b[...] + recv_b[1, :, :]
        b2 = stage(2, acc_b, recv_b, send_b, recv_sem_b, order_b[2])
        b2.start()

        a2.wait()
        acc_a[...] = acc_a[...] + recv_a[2, :, :]
        finish(0, acc_a)
        b2.wait()
        acc_b[...] = acc_b[...] + recv_b[2, :, :]
        finish(1, acc_b)

    return pl.pallas_call(
        body,
        out_shape=jax.ShapeDtypeStruct((B, SQ, 512), jnp.float32),
        in_specs=[pl.BlockSpec(memory_space=pltpu.VMEM)] * 5,
        out_specs=pl.BlockSpec(memory_space=pltpu.VMEM),
        scratch_shapes=[
            pltpu.VMEM((SQ, PC), BF),
            pltpu.VMEM((SQ, PC), BF),
            pltpu.VMEM((3, SQ, PC), BF),
            pltpu.VMEM((3, SQ, PC), BF),
            pltpu.SemaphoreType.DMA((3,)),
            pltpu.SemaphoreType.DMA((3,)),
            pltpu.SemaphoreType.DMA((3,)),
            pltpu.SemaphoreType.DMA((3,)),
        ],
        compiler_params=pltpu.CompilerParams(collective_id=0),
    )(x, Wq, K_ext, V_ext, Wo)


# device time: 16660 ns/iter; 1.0612x vs baseline; 1.0612x over previous
import jax
import jax.numpy as jnp
from jax import lax
from jax.experimental import pallas as pl
from jax.experimental.pallas import tpu as pltpu

N_DEV = 8
B, SQ, H, D = 2, 128, 4, 64
HD = H * D
ROWS = B * SQ
PR = 136
NEG = -1e9
BF = jnp.bfloat16


def kernel(x, Wq, K_ext, V_ext, Wo):
    def body(x_ref, wq_ref, k_ref, v_ref, wo_ref, out_ref,
             acc_a, acc_b, recv_a, recv_b,
             send_a, recv_sem_a, send_b, recv_sem_b):
        my = lax.axis_index("i")
        p_x = jnp.bitwise_xor(my, 1)
        loc = lax.rem(my, 4)
        p_y = my - loc + (3 - loc)
        p_z = jnp.bitwise_xor(my, 4)
        order_a = [p_x, p_y, p_z]
        order_b = [p_y, p_z, p_x]

        barrier_sem = pltpu.get_barrier_semaphore()
        for nbr in (p_x, p_y, p_z):
            pl.semaphore_signal(
                barrier_sem, inc=1,
                device_id=(nbr,), device_id_type=pl.DeviceIdType.MESH,
            )

        x2 = x_ref[...].reshape(ROWS, 512).astype(BF)
        q2 = jnp.dot(x2, wq_ref[...].astype(BF),
                     preferred_element_type=jnp.float32).astype(BF)
        k2 = k_ref[...].reshape(ROWS, HD).astype(BF)
        v2 = v_ref[...].reshape(ROWS, HD).astype(BF)

        ri = lax.broadcasted_iota(jnp.int32, (SQ, SQ), 0) // 64
        ci = lax.broadcasted_iota(jnp.int32, (SQ, SQ), 1) // 64
        blockdiag = ri == ci
        is_even = lax.rem(my, 2) == 0

        def partial(b, acc):
            for hh in range(H):
                q = q2[b * SQ:(b + 1) * SQ, hh * D:(hh + 1) * D]
                kc = k2[b * SQ:(b + 1) * SQ, hh * D:(hh + 1) * D]
                vc = v2[b * SQ:(b + 1) * SQ, hh * D:(hh + 1) * D]
                s_t = lax.dot_general(
                    kc, q, (((1,), (1,)), ((), ())),
                    preferred_element_type=jnp.float32,
                ) * 0.125
                p_t = jnp.exp(jnp.where(blockdiag, s_t, NEG))
                p_t = jnp.where(is_even, p_t, 0.0)
                l_row = jnp.sum(p_t, axis=0, keepdims=True)
                o_bh = lax.dot_general(
                    p_t.astype(BF), vc, (((0,), (0,)), ((), ())),
                    preferred_element_type=jnp.float32,
                )
                acc[0:SQ, hh * D:(hh + 1) * D] = o_bh.astype(BF)
                acc[SQ + hh:SQ + hh + 1, 0:SQ] = l_row.astype(BF)
                acc[SQ + hh:SQ + hh + 1, SQ:HD] = jnp.zeros((1, SQ), BF)

        def stage(s, acc, recv, ssem, rsem, partner):
            return pltpu.make_async_remote_copy(
                src_ref=acc,
                dst_ref=recv.at[s],
                send_sem=ssem.at[s],
                recv_sem=rsem.at[s],
                device_id=(partner,),
                device_id_type=pl.DeviceIdType.MESH,
            )

        def finish(b, acc, recv):
            tot = (acc[...].astype(jnp.float32)
                   + recv[2, :, :].astype(jnp.float32))
            o2 = tot[0:SQ, :]
            heads = []
            for hh in range(H):
                l_col = tot[SQ + hh:SQ + hh + 1, 0:SQ].reshape(SQ, 1)
                heads.append(o2[:, hh * D:(hh + 1) * D] / l_col)
            ctx = jnp.concatenate(heads, axis=1)
            out_b = jnp.dot(ctx.astype(BF), wo_ref[...].astype(BF),
                            preferred_element_type=jnp.float32)
            out_ref[b, :, :] = out_b

        partial(0, acc_a)
        pl.semaphore_wait(barrier_sem, 3)
        a0 = stage(0, acc_a, recv_a, send_a, recv_sem_a, order_a[0])
        a0.start()
        partial(1, acc_b)
        b0 = stage(0, acc_b, recv_b, send_b, recv_sem_b, order_b[0])
        b0.start()

        a0.wait()
        acc_a[...] = acc_a[...] + recv_a[0, :, :]
        a1 = stage(1, acc_a, recv_a, send_a, recv_sem_a, order_a[1])
        a1.start()
        b0.wait()
        acc_b[...] = acc_b[...] + recv_b[0, :, :]
        b1 = stage(1, acc_b, recv_b, send_b, recv_sem_b, order_b[1])
        b1.start()

        a1.wait()
        acc_a[...] = acc_a[...] + recv_a[1, :, :]
        a2 = stage(2, acc_a, recv_a, send_a, recv_sem_a, order_a[2])
        a2.start()
        b1.wait()
        acc_b[...] = acc_b[...] + recv_b[1, :, :]
        b2 = stage(2, acc_b, recv_b, send_b, recv_sem_b, order_b[2])
        b2.start()

        a2.wait()
        finish(0, acc_a, recv_a)
        b2.wait()
        finish(1, acc_b, recv_b)

    return pl.pallas_call(
        body,
        out_shape=jax.ShapeDtypeStruct((B, SQ, 512), jnp.float32),
        in_specs=[pl.BlockSpec(memory_space=pltpu.VMEM)] * 5,
        out_specs=pl.BlockSpec(memory_space=pltpu.VMEM),
        scratch_shapes=[
            pltpu.VMEM((PR, HD), BF),
            pltpu.VMEM((PR, HD), BF),
            pltpu.VMEM((3, PR, HD), BF),
            pltpu.VMEM((3, PR, HD), BF),
            pltpu.SemaphoreType.DMA((3,)),
            pltpu.SemaphoreType.DMA((3,)),
            pltpu.SemaphoreType.DMA((3,)),
            pltpu.SemaphoreType.DMA((3,)),
        ],
        compiler_params=pltpu.CompilerParams(collective_id=0),
    )(x, Wq, K_ext, V_ext, Wo)


# device time: 15941 ns/iter; 1.1091x vs baseline; 1.0451x over previous
import jax
import jax.numpy as jnp
from jax import lax
from jax.experimental import pallas as pl
from jax.experimental.pallas import tpu as pltpu

N_DEV = 8
B, SQ, H, D = 2, 128, 4, 64
HD = H * D
ROWS = B * SQ
PR = 136
NEG = -1e9
BF = jnp.bfloat16


def kernel(x, Wq, K_ext, V_ext, Wo):
    def body(x_ref, wq_ref, k_ref, v_ref, wo_ref, out_ref,
             acc_a, acc_b, recv_a, recv_b,
             send_a, recv_sem_a, send_b, recv_sem_b):
        my = lax.axis_index("i")
        p_x = jnp.bitwise_xor(my, 1)
        loc = lax.rem(my, 4)
        base = my - loc
        p_y = base + (3 - loc)
        p_diag = base + lax.rem(loc + 2, 4)
        p_z = jnp.bitwise_xor(my, 4)
        peers = [p_x, p_y, p_diag, p_z]

        barrier_sem = pltpu.get_barrier_semaphore()
        for nbr in peers:
            pl.semaphore_signal(
                barrier_sem, inc=1,
                device_id=(nbr,), device_id_type=pl.DeviceIdType.MESH,
            )

        x2 = x_ref[...].reshape(ROWS, 512).astype(BF)
        q2 = (jnp.dot(x2, wq_ref[...].astype(BF),
                      preferred_element_type=jnp.float32)
              * 0.125).astype(BF)
        k2 = k_ref[...].reshape(ROWS, HD).astype(BF)
        v2 = v_ref[...].reshape(ROWS, HD).astype(BF)

        ri = lax.broadcasted_iota(jnp.int32, (SQ, SQ), 0) // 64
        ci = lax.broadcasted_iota(jnp.int32, (SQ, SQ), 1) // 64
        blockdiag = ri == ci
        is_even = lax.rem(my, 2) == 0

        def partial(b, acc):
            for hh in range(H):
                q = q2[b * SQ:(b + 1) * SQ, hh * D:(hh + 1) * D]
                kc = k2[b * SQ:(b + 1) * SQ, hh * D:(hh + 1) * D]
                vc = v2[b * SQ:(b + 1) * SQ, hh * D:(hh + 1) * D]
                s_t = lax.dot_general(
                    kc, q, (((1,), (1,)), ((), ())),
                    preferred_element_type=jnp.float32,
                )
                p_t = jnp.exp(jnp.where(blockdiag, s_t, NEG))
                p_t = jnp.where(is_even, p_t, 0.0)
                l_row = jnp.sum(p_t, axis=0, keepdims=True)
                o_bh = lax.dot_general(
                    p_t.astype(BF), vc, (((0,), (0,)), ((), ())),
                    preferred_element_type=jnp.float32,
                )
                acc[0:SQ, hh * D:(hh + 1) * D] = o_bh.astype(BF)
                acc[SQ + hh:SQ + hh + 1, 0:SQ] = l_row.astype(BF)
                acc[SQ + hh:SQ + hh + 1, SQ:HD] = jnp.zeros((1, SQ), BF)

        def rdma(slot, acc, recv, ssem, rsem, peer):
            return pltpu.make_async_remote_copy(
                src_ref=acc,
                dst_ref=recv.at[slot],
                send_sem=ssem.at[slot],
                recv_sem=rsem.at[slot],
                device_id=(peer,),
                device_id_type=pl.DeviceIdType.MESH,
            )

        def plane_start(acc, recv, ssem, rsem):
            rs = [rdma(i, acc, recv, ssem, rsem, peers[i]) for i in range(3)]
            for r in rs:
                r.start()
            return rs

        def plane_merge(rs, acc, recv):
            for r in rs:
                r.wait()
            acc[...] = (acc[...] + recv[0, :, :]
                        + recv[1, :, :] + recv[2, :, :])

        def finish(b, acc, recv):
            tot = (acc[...].astype(jnp.float32)
                   + recv[3, :, :].astype(jnp.float32))
            o2 = tot[0:SQ, :]
            heads = []
            for hh in range(H):
                l_col = tot[SQ + hh:SQ + hh + 1, 0:SQ].reshape(SQ, 1)
                heads.append(o2[:, hh * D:(hh + 1) * D] / l_col)
            ctx = jnp.concatenate(heads, axis=1)
            out_b = jnp.dot(ctx.astype(BF), wo_ref[...].astype(BF),
                            preferred_element_type=jnp.float32)
            out_ref[b, :, :] = out_b

        partial(0, acc_a)
        pl.semaphore_wait(barrier_sem, 4)
        ra = plane_start(acc_a, recv_a, send_a, recv_sem_a)
        partial(1, acc_b)
        rb = plane_start(acc_b, recv_b, send_b, recv_sem_b)

        plane_merge(ra, acc_a, recv_a)
        az = rdma(3, acc_a, recv_a, send_a, recv_sem_a, p_z)
        az.start()
        plane_merge(rb, acc_b, recv_b)
        bz = rdma(3, acc_b, recv_b, send_b, recv_sem_b, p_z)
        bz.start()

        az.wait()
        finish(0, acc_a, recv_a)
        bz.wait()
        finish(1, acc_b, recv_b)

    return pl.pallas_call(
        body,
        out_shape=jax.ShapeDtypeStruct((B, SQ, 512), jnp.float32),
        in_specs=[pl.BlockSpec(memory_space=pltpu.VMEM)] * 5,
        out_specs=pl.BlockSpec(memory_space=pltpu.VMEM),
        scratch_shapes=[
            pltpu.VMEM((PR, HD), BF),
            pltpu.VMEM((PR, HD), BF),
            pltpu.VMEM((4, PR, HD), BF),
            pltpu.VMEM((4, PR, HD), BF),
            pltpu.SemaphoreType.DMA((4,)),
            pltpu.SemaphoreType.DMA((4,)),
            pltpu.SemaphoreType.DMA((4,)),
            pltpu.SemaphoreType.DMA((4,)),
        ],
        compiler_params=pltpu.CompilerParams(collective_id=0),
    )(x, Wq, K_ext, V_ext, Wo)
